# Initial kernel scaffold; baseline (speedup 1.0000x reference)
import jax
import jax.numpy as jnp
from jax.experimental import pallas as pl


def kernel(position_mask, pe_k, pe_v):
    raise NotImplementedError("write your pallas kernel here")


# speedup vs baseline: 4.3334x; 4.3334x over previous
import functools

import jax
import jax.numpy as jnp
from jax import lax
from jax.experimental import pallas as pl
from jax.experimental.pallas import tpu as pltpu
from jax.experimental.pallas import tpu_sc as plsc

MAX_LEN = 200
D = 32
N = 4096 * 200
SEG = 128
CHUNK = 512
L = 16


@functools.lru_cache(maxsize=None)
def _build_lookup():
    info = plsc.get_sparse_core_info()
    nc, ns = info.num_cores, info.num_subcores
    nw = nc * ns
    n_per_w = N // nw
    n_chunks = n_per_w // CHUNK
    segs = CHUNK // SEG
    mesh = plsc.VectorSubcoreMesh(core_axis_name="c", subcore_axis_name="s")

    @functools.partial(
        pl.kernel,
        mesh=mesh,
        compiler_params=pltpu.CompilerParams(use_tc_tiling_on_sc=False),
        out_type=[
            jax.ShapeDtypeStruct((N, D), jnp.float32),
            jax.ShapeDtypeStruct((N, D), jnp.float32),
        ],
        scratch_types=[
            pltpu.VMEM((segs, SEG), jnp.int32),
            pltpu.VMEM((CHUNK, D), jnp.float32),
            pltpu.VMEM((CHUNK, D), jnp.float32),
            pltpu.SemaphoreType.DMA,
            pltpu.SemaphoreType.DMA,
        ],
    )
    def lookup(idx_hbm, pek_hbm, pev_hbm, outk_hbm, outv_hbm,
               idx_v, krows, vrows, gsem, osem):
        wid = lax.axis_index("s") * nc + lax.axis_index("c")
        idx_row0 = wid * (n_per_w // SEG)
        out_row0 = wid * n_per_w

        def chunk_body(g, carry):
            pltpu.sync_copy(idx_hbm.at[pl.ds(idx_row0 + g * segs, segs)], idx_v)
            for j in range(segs):
                for i in range(SEG // L):
                    sl = (j, pl.ds(i * L, L))
                    v = idx_v[sl]
                    idx_v[sl] = jnp.minimum(jnp.maximum(v, 0), MAX_LEN)
            copies = []
            for j in range(segs):
                copies.append(pltpu.async_copy(
                    pek_hbm.at[idx_v.at[j]], krows.at[pl.ds(j * SEG, SEG)], gsem))
                copies.append(pltpu.async_copy(
                    pev_hbm.at[idx_v.at[j]], vrows.at[pl.ds(j * SEG, SEG)], gsem))
            for c in copies:
                c.wait()
            out_base = out_row0 + g * CHUNK
            ck = pltpu.async_copy(krows, outk_hbm.at[pl.ds(out_base, CHUNK)], osem)
            cv = pltpu.async_copy(vrows, outv_hbm.at[pl.ds(out_base, CHUNK)], osem)
            ck.wait()
            cv.wait()
            return carry

        lax.fori_loop(0, n_chunks, chunk_body, 0)

    return lookup


def kernel(position_mask, pe_k, pe_v):
    idx = position_mask.reshape(-1).astype(jnp.int32).reshape(N // SEG, SEG)
    out_k, out_v = _build_lookup()(idx, pe_k, pe_v)
    b, s = position_mask.shape
    return out_k.reshape(b, s, D), out_v.reshape(b, s, D)


# speedup vs baseline: 4.3857x; 1.0121x over previous
import functools

import jax
import jax.numpy as jnp
from jax import lax
from jax.experimental import pallas as pl
from jax.experimental.pallas import tpu as pltpu
from jax.experimental.pallas import tpu_sc as plsc

MAX_LEN = 200
D = 32
N = 4096 * 200
SEG = 128
CHUNK = 512
L = 16
SEGS = CHUNK // SEG


@functools.lru_cache(maxsize=None)
def _build_lookup():
    info = plsc.get_sparse_core_info()
    nc, ns = info.num_cores, info.num_subcores
    nw = nc * ns
    n_per_w = N // nw
    n_chunks = n_per_w // CHUNK
    assert n_chunks % 2 == 0 and n_chunks >= 4
    mesh = plsc.VectorSubcoreMesh(core_axis_name="c", subcore_axis_name="s")

    @functools.partial(
        pl.kernel,
        mesh=mesh,
        compiler_params=pltpu.CompilerParams(use_tc_tiling_on_sc=False),
        out_type=[
            jax.ShapeDtypeStruct((N, D), jnp.float32),
            jax.ShapeDtypeStruct((N, D), jnp.float32),
        ],
        scratch_types=[
            pltpu.VMEM((2, SEGS, SEG), jnp.int32),
            pltpu.VMEM((2, CHUNK, D), jnp.float32),
            pltpu.VMEM((2, CHUNK, D), jnp.float32),
            pltpu.SemaphoreType.DMA,
            pltpu.SemaphoreType.DMA,
            pltpu.SemaphoreType.DMA,
        ],
    )
    def lookup(idx_hbm, pek_hbm, pev_hbm, outk_hbm, outv_hbm,
               idx_v, krows, vrows, gsem, isem, osem):
        wid = lax.axis_index("s") * nc + lax.axis_index("c")
        idx_row0 = wid * (n_per_w // SEG)
        out_row0 = wid * n_per_w
        last_pf = (n_chunks - 1) * SEGS

        def prefetch_idx(c, p):
            row = idx_row0 + jnp.minimum(c * SEGS, last_pf)
            pltpu.async_copy(idx_hbm.at[pl.ds(row, SEGS)], idx_v.at[p], isem)

        def wait_idx(p):
            pltpu.make_async_copy(
                idx_hbm.at[pl.ds(0, SEGS)], idx_v.at[p], isem).wait()

        def clip(p):
            for j in range(SEGS):
                for i in range(SEG // L):
                    sl = (p, j, pl.ds(i * L, L))
                    idx_v[sl] = jnp.minimum(jnp.maximum(idx_v[sl], 0), MAX_LEN)

        def drain_scatter_pair():
            pltpu.make_async_copy(
                krows.at[0], outk_hbm.at[pl.ds(0, CHUNK)], osem).wait()
            pltpu.make_async_copy(
                vrows.at[0], outv_hbm.at[pl.ds(0, CHUNK)], osem).wait()

        def step(c, p, drain):
            wait_idx(p)
            clip(p)
            if drain:
                drain_scatter_pair()
            copies = []
            for j in range(SEGS):
                copies.append(pltpu.async_copy(
                    pek_hbm.at[idx_v.at[p, j]],
                    krows.at[p, pl.ds(j * SEG, SEG)], gsem))
                copies.append(pltpu.async_copy(
                    pev_hbm.at[idx_v.at[p, j]],
                    vrows.at[p, pl.ds(j * SEG, SEG)], gsem))
            prefetch_idx(c + 1, 1 - p)
            for cp in copies:
                cp.wait()
            base = out_row0 + c * CHUNK
            pltpu.async_copy(krows.at[p], outk_hbm.at[pl.ds(base, CHUNK)], osem)
            pltpu.async_copy(vrows.at[p], outv_hbm.at[pl.ds(base, CHUNK)], osem)

        prefetch_idx(0, 0)
        step(0, 0, drain=False)
        step(1, 1, drain=False)

        def pair_body(i, carry):
            step(2 * i + 2, 0, drain=True)
            step(2 * i + 3, 1, drain=True)
            return carry

        lax.fori_loop(0, (n_chunks - 2) // 2, pair_body, 0)
        wait_idx(0)
        drain_scatter_pair()
        drain_scatter_pair()

    return lookup


def kernel(position_mask, pe_k, pe_v):
    idx = position_mask.reshape(-1).astype(jnp.int32).reshape(N // SEG, SEG)
    out_k, out_v = _build_lookup()(idx, pe_k, pe_v)
    b, s = position_mask.shape
    return out_k.reshape(b, s, D), out_v.reshape(b, s, D)


# speedup vs baseline: 6.8499x; 1.5619x over previous
import functools

import jax
import jax.numpy as jnp
from jax import lax
from jax.experimental import pallas as pl
from jax.experimental.pallas import tpu as pltpu
from jax.experimental.pallas import tpu_sc as plsc

MAX_LEN = 200
D = 32
B = 4096
S = 200
L = 16
NB = 2
QB = 8
SUBS = QB // NB
CLIP_OFFS = tuple(range(0, S - L + 1, L)) + (S - L,)


@functools.lru_cache(maxsize=None)
def _build_lookup():
    info = plsc.get_sparse_core_info()
    nc, ns = info.num_cores, info.num_subcores
    nw = nc * ns
    rows_per_w = B // nw
    n_q = rows_per_w // QB
    mesh = plsc.VectorSubcoreMesh(core_axis_name="c", subcore_axis_name="s")

    @functools.partial(
        pl.kernel,
        mesh=mesh,
        compiler_params=pltpu.CompilerParams(use_tc_tiling_on_sc=False),
        out_type=[
            jax.ShapeDtypeStruct((B, S, D), jnp.float32),
            jax.ShapeDtypeStruct((B, S, D), jnp.float32),
        ],
        scratch_types=[
            pltpu.VMEM((2, QB, S), jnp.int32),
            pltpu.VMEM((2, NB, S, D), jnp.float32),
            pltpu.VMEM((2, NB, S, D), jnp.float32),
            pltpu.VMEM_SHARED((MAX_LEN + 1, D), jnp.float32),
            pltpu.VMEM_SHARED((MAX_LEN + 1, D), jnp.float32),
            pltpu.SemaphoreType.DMA,
            pltpu.SemaphoreType.DMA,
            pltpu.SemaphoreType.DMA,
        ],
    )
    def lookup(idx_hbm, pek_hbm, pev_hbm, outk_hbm, outv_hbm,
               idx_v, krows, vrows, ktbl_sh, vtbl_sh, gsem, isem, osem):
        cid = lax.axis_index("c")
        sid = lax.axis_index("s")
        wid = sid * nc + cid
        b0 = wid * rows_per_w

        @pl.when(sid == 0)
        def _stage():
            pltpu.sync_copy(pek_hbm, ktbl_sh)
            pltpu.sync_copy(pev_hbm, vtbl_sh)
        plsc.subcore_barrier()

        def prefetch_idx(q, p):
            row = b0 + jnp.minimum(q, n_q - 1) * QB
            pltpu.async_copy(idx_hbm.at[pl.ds(row, QB)], idx_v.at[p], isem)

        def wait_idx(p):
            pltpu.make_async_copy(
                idx_hbm.at[pl.ds(0, QB)], idx_v.at[p], isem).wait()

        def clip(p):
            for r in range(QB):
                for off in CLIP_OFFS:
                    sl = (p, r, pl.ds(off, L))
                    idx_v[sl] = jnp.minimum(jnp.maximum(idx_v[sl], 0), MAX_LEN)

        def drain_scatter_pair():
            pltpu.make_async_copy(
                krows.at[0], outk_hbm.at[pl.ds(0, NB)], osem).wait()
            pltpu.make_async_copy(
                vrows.at[0], outv_hbm.at[pl.ds(0, NB)], osem).wait()

        def sub_chunk(q, qp, u, drain):
            p = u % 2
            if drain:
                drain_scatter_pair()
            copies = []
            for rr in range(NB):
                r = u * NB + rr
                for off, n in ((0, 128), (128, S - 128)):
                    idx_sl = idx_v.at[qp, r, pl.ds(off, n)]
                    copies.append(pltpu.async_copy(
                        ktbl_sh.at[idx_sl], krows.at[p, rr, pl.ds(off, n)],
                        gsem))
                    copies.append(pltpu.async_copy(
                        vtbl_sh.at[idx_sl], vrows.at[p, rr, pl.ds(off, n)],
                        gsem))
            for cp in copies:
                cp.wait()
            out_b = b0 + q * QB + u * NB
            pltpu.async_copy(krows.at[p], outk_hbm.at[pl.ds(out_b, NB)], osem)
            pltpu.async_copy(vrows.at[p], outv_hbm.at[pl.ds(out_b, NB)], osem)

        prefetch_idx(0, 0)
        wait_idx(0)
        clip(0)
        prefetch_idx(1, 1)
        for u in range(SUBS):
            sub_chunk(0, 0, u, drain=(u >= 2))

        def q_body(i, carry):
            q = 2 * i + 1
            for qq, qp in ((q, 1), (q + 1, 0)):
                wait_idx(qp)
                clip(qp)
                prefetch_idx(qq + 1, 1 - qp)
                for u in range(SUBS):
                    sub_chunk(qq, qp, u, drain=True)
            return carry

        lax.fori_loop(0, (n_q - 1) // 2, q_body, 0)
        wait_idx(1)
        clip(1)
        prefetch_idx(n_q, 0)
        for u in range(SUBS):
            sub_chunk(n_q - 1, 1, u, drain=True)
        wait_idx(0)
        drain_scatter_pair()
        drain_scatter_pair()

    return lookup


def kernel(position_mask, pe_k, pe_v):
    idx = position_mask.astype(jnp.int32)
    return _build_lookup()(idx, pe_k, pe_v)


# speedup vs baseline: 6.8749x; 1.0037x over previous
import functools

import jax
import jax.numpy as jnp
from jax import lax
from jax.experimental import pallas as pl
from jax.experimental.pallas import tpu as pltpu
from jax.experimental.pallas import tpu_sc as plsc

MAX_LEN = 200
D = 32
B = 4096
S = 200
L = 16
NB = 2
QB = 8
SUBS = QB // NB
CLIP_OFFS = tuple(range(0, S - L + 1, L)) + (S - L,)


@functools.lru_cache(maxsize=None)
def _build_lookup():
    info = plsc.get_sparse_core_info()
    nc, ns = info.num_cores, info.num_subcores
    nw = nc * ns
    rows_per_w = B // nw
    n_q = rows_per_w // QB
    mesh = plsc.VectorSubcoreMesh(core_axis_name="c", subcore_axis_name="s")

    @functools.partial(
        pl.kernel,
        mesh=mesh,
        compiler_params=pltpu.CompilerParams(use_tc_tiling_on_sc=False),
        out_type=[
            jax.ShapeDtypeStruct((B, S, D), jnp.float32),
            jax.ShapeDtypeStruct((B, S, D), jnp.float32),
        ],
        scratch_types=[
            pltpu.VMEM((2, QB, S), jnp.int32),
            pltpu.VMEM((2, NB, S, D), jnp.float32),
            pltpu.VMEM((2, NB, S, D), jnp.float32),
            pltpu.VMEM_SHARED((MAX_LEN + 1, D), jnp.float32),
            pltpu.VMEM_SHARED((MAX_LEN + 1, D), jnp.float32),
            pltpu.SemaphoreType.DMA,
            pltpu.SemaphoreType.DMA,
            pltpu.SemaphoreType.DMA,
        ],
    )
    def lookup(idx_hbm, pek_hbm, pev_hbm, outk_hbm, outv_hbm,
               idx_v, krows, vrows, ktbl_sh, vtbl_sh, gsem, isem, osem):
        cid = lax.axis_index("c")
        sid = lax.axis_index("s")
        wid = sid * nc + cid
        b0 = wid * rows_per_w

        @pl.when(sid == 0)
        def _stage():
            pltpu.sync_copy(pek_hbm, ktbl_sh)
            pltpu.sync_copy(pev_hbm, vtbl_sh)
        plsc.subcore_barrier()

        def prefetch_idx(q, p):
            row = b0 + jnp.minimum(q, n_q - 1) * QB
            pltpu.async_copy(idx_hbm.at[pl.ds(row, QB)], idx_v.at[p], isem)

        def wait_idx(p):
            pltpu.make_async_copy(
                idx_hbm.at[pl.ds(0, QB)], idx_v.at[p], isem).wait()

        def clip(p):
            for r in range(QB):
                for off in CLIP_OFFS:
                    sl = (p, r, pl.ds(off, L))
                    idx_v[sl] = jnp.minimum(jnp.maximum(idx_v[sl], 0), MAX_LEN)

        def drain_scatter_pair():
            pltpu.make_async_copy(
                krows.at[0], outk_hbm.at[pl.ds(0, NB)], osem).wait()
            pltpu.make_async_copy(
                vrows.at[0], outv_hbm.at[pl.ds(0, NB)], osem).wait()

        def sub_chunk(q, qp, u, drain):
            p = u % 2
            if drain:
                drain_scatter_pair()
            copies = []
            for rr in range(NB):
                r = u * NB + rr
                for off, n in ((0, 128), (128, S - 128)):
                    idx_sl = idx_v.at[qp, r, pl.ds(off, n)]
                    copies.append(pltpu.async_copy(
                        ktbl_sh.at[idx_sl], krows.at[p, rr, pl.ds(off, n)],
                        gsem))
                    copies.append(pltpu.async_copy(
                        vtbl_sh.at[idx_sl], vrows.at[p, rr, pl.ds(off, n)],
                        gsem))
            for cp in copies:
                cp.wait()
            out_b = b0 + q * QB + u * NB
            pltpu.async_copy(krows.at[p], outk_hbm.at[pl.ds(out_b, NB)], osem)
            pltpu.async_copy(vrows.at[p], outv_hbm.at[pl.ds(out_b, NB)], osem)

        prefetch_idx(0, 0)
        wait_idx(0)
        clip(0)
        prefetch_idx(1, 1)
        for u in range(SUBS):
            sub_chunk(0, 0, u, drain=(u >= 2))

        def q_body(i, carry):
            q = 2 * i + 1
            for qq, qp in ((q, 1), (q + 1, 0)):
                wait_idx(qp)
                clip(qp)
                prefetch_idx(qq + 1, 1 - qp)
                for u in range(SUBS):
                    sub_chunk(qq, qp, u, drain=True)
            return carry

        lax.fori_loop(0, (n_q - 1) // 2, q_body, 0)
        wait_idx(1)
        clip(1)
        prefetch_idx(n_q, 0)
        for u in range(SUBS):
            sub_chunk(n_q - 1, 1, u, drain=True)
        wait_idx(0)
        drain_scatter_pair()
        drain_scatter_pair()

    return lookup


def kernel(position_mask, pe_k, pe_v):
    idx = position_mask.astype(jnp.int32)
    out_k, out_v = _build_lookup()(idx, pe_k, pe_v)
    return (out_k, out_v)


# speedup vs baseline: 8.7953x; 1.2793x over previous
import functools

import jax
import jax.numpy as jnp
from jax import lax
from jax.experimental import pallas as pl
from jax.experimental.pallas import tpu as pltpu
from jax.experimental.pallas import tpu_sc as plsc

MAX_LEN = 200
D = 32
W = 128
B = 4096
S = 200
L = 16
QB = 8
CLIP_OFFS = tuple(range(0, S - L + 1, L)) + (S - L,)


@functools.lru_cache(maxsize=None)
def _build_lookup():
    info = plsc.get_sparse_core_info()
    nc, ns = info.num_cores, info.num_subcores
    nw = nc * ns
    rows_per_w = B // nw
    n_q = rows_per_w // QB
    mesh = plsc.VectorSubcoreMesh(core_axis_name="c", subcore_axis_name="s")

    @functools.partial(
        pl.kernel,
        mesh=mesh,
        out_type=[
            jax.ShapeDtypeStruct((B, S, W), jnp.float32),
            jax.ShapeDtypeStruct((B, S, W), jnp.float32),
        ],
        scratch_types=[
            pltpu.VMEM((2, QB, S), jnp.int32),
            pltpu.VMEM((2, 1, S, W), jnp.float32),
            pltpu.VMEM((2, 1, S, W), jnp.float32),
            pltpu.VMEM_SHARED((MAX_LEN + 1, W), jnp.float32),
            pltpu.VMEM_SHARED((MAX_LEN + 1, W), jnp.float32),
            pltpu.SemaphoreType.DMA,
            pltpu.SemaphoreType.DMA,
            pltpu.SemaphoreType.DMA,
        ],
    )
    def lookup(idx_hbm, tblk_hbm, tblv_hbm, outk_hbm, outv_hbm,
               idx_v, krows, vrows, ktbl_sh, vtbl_sh, gsem, isem, osem):
        cid = lax.axis_index("c")
        sid = lax.axis_index("s")
        wid = sid * nc + cid
        b0 = wid * rows_per_w

        @pl.when(sid == 0)
        def _stage():
            pltpu.sync_copy(tblk_hbm, ktbl_sh)
            pltpu.sync_copy(tblv_hbm, vtbl_sh)
        plsc.subcore_barrier()

        def prefetch_idx(q, p):
            row = b0 + jnp.minimum(q, n_q - 1) * QB
            pltpu.async_copy(idx_hbm.at[pl.ds(row, QB)], idx_v.at[p], isem)

        def wait_idx(p):
            pltpu.make_async_copy(
                idx_hbm.at[pl.ds(0, QB)], idx_v.at[p], isem).wait()

        def clip(p):
            for r in range(QB):
                for off in CLIP_OFFS:
                    sl = (p, r, pl.ds(off, L))
                    idx_v[sl] = jnp.minimum(jnp.maximum(idx_v[sl], 0), MAX_LEN)

        def drain_scatter_pair():
            pltpu.make_async_copy(
                krows.at[0], outk_hbm.at[pl.ds(0, 1)], osem).wait()
            pltpu.make_async_copy(
                vrows.at[0], outv_hbm.at[pl.ds(0, 1)], osem).wait()

        def sub_chunk(q, qp, u, drain):
            p = u % 2
            if drain:
                drain_scatter_pair()
            copies = []
            for off, n in ((0, 128), (128, S - 128)):
                idx_sl = idx_v.at[qp, u, pl.ds(off, n)]
                copies.append(pltpu.async_copy(
                    ktbl_sh.at[idx_sl], krows.at[p, 0, pl.ds(off, n)], gsem))
                copies.append(pltpu.async_copy(
                    vtbl_sh.at[idx_sl], vrows.at[p, 0, pl.ds(off, n)], gsem))
            for cp in copies:
                cp.wait()
            out_b = b0 + q * QB + u
            pltpu.async_copy(krows.at[p], outk_hbm.at[pl.ds(out_b, 1)], osem)
            pltpu.async_copy(vrows.at[p], outv_hbm.at[pl.ds(out_b, 1)], osem)

        prefetch_idx(0, 0)
        wait_idx(0)
        clip(0)
        prefetch_idx(1, 1)
        for u in range(QB):
            sub_chunk(0, 0, u, drain=(u >= 2))

        def q_body(i, carry):
            q = 2 * i + 1
            for qq, qp in ((q, 1), (q + 1, 0)):
                wait_idx(qp)
                clip(qp)
                prefetch_idx(qq + 1, 1 - qp)
                for u in range(QB):
                    sub_chunk(qq, qp, u, drain=True)
            return carry

        lax.fori_loop(0, (n_q - 1) // 2, q_body, 0)
        wait_idx(1)
        clip(1)
        prefetch_idx(n_q, 0)
        for u in range(QB):
            sub_chunk(n_q - 1, 1, u, drain=True)
        wait_idx(0)
        drain_scatter_pair()
        drain_scatter_pair()

    return lookup


def kernel(position_mask, pe_k, pe_v):
    idx = position_mask.astype(jnp.int32)
    pad = jnp.zeros((MAX_LEN + 1, W - 2 * D), jnp.float32)
    tblk = jnp.concatenate([pe_k, pe_v, pad], axis=1)
    tblv = jnp.concatenate([pe_v, pe_k, pad], axis=1)
    out_k, out_v = _build_lookup()(idx, tblk, tblv)
    return (out_k[:, :, :D], out_v[:, :, :D])


# speedup vs baseline: 9.6014x; 1.0917x over previous
import functools

import jax
import jax.numpy as jnp
from jax import lax
from jax.experimental import pallas as pl
from jax.experimental.pallas import tpu as pltpu
from jax.experimental.pallas import tpu_sc as plsc

MAX_LEN = 200
D = 32
B = 4096
S = 200
L = 16
TW = 256


@functools.lru_cache(maxsize=None)
def _build_lookup():
    info = plsc.get_sparse_core_info()
    nc, ns = info.num_cores, info.num_subcores
    nw = nc * ns
    lanes_per_w = B // nw
    mesh = plsc.VectorSubcoreMesh(core_axis_name="c", subcore_axis_name="s")

    @functools.partial(
        pl.kernel,
        mesh=mesh,
        compiler_params=pltpu.CompilerParams(needs_layout_passes=False),
        out_type=[
            jax.ShapeDtypeStruct((S, D, B), jnp.float32),
            jax.ShapeDtypeStruct((S, D, B), jnp.float32),
        ],
        scratch_types=[
            pltpu.VMEM((D * TW,), jnp.float32),
            pltpu.VMEM((D * TW,), jnp.float32),
            pltpu.VMEM((2, 128), jnp.int32),
            pltpu.VMEM((2, D, 128), jnp.float32),
            pltpu.VMEM((2, D, 128), jnp.float32),
            pltpu.SemaphoreType.DMA,
            pltpu.SemaphoreType.DMA,
            pltpu.SemaphoreType.DMA,
        ],
    )
    def lookup(idx_hbm, tblk_hbm, tblv_hbm, outk_hbm, outv_hbm,
               ktbl, vtbl, ibuf, kobuf, vobuf, tsem, isem, osem):
        wid = lax.axis_index("s") * nc + lax.axis_index("c")
        b0 = wid * lanes_per_w

        ct = pltpu.async_copy(tblk_hbm, ktbl, tsem)
        cv = pltpu.async_copy(tblv_hbm, vtbl, tsem)
        ct.wait()
        cv.wait()

        def prefetch_idx(s, p):
            row = jnp.minimum(s, S - 1)
            pltpu.async_copy(idx_hbm.at[row, pl.ds(b0, 128)], ibuf.at[p],
                             isem)

        def wait_idx(p):
            pltpu.make_async_copy(
                idx_hbm.at[0, pl.ds(0, 128)], ibuf.at[p], isem).wait()

        def drain_out_pair():
            pltpu.make_async_copy(
                kobuf.at[0], outk_hbm.at[0, :, pl.ds(0, 128)], osem).wait()
            pltpu.make_async_copy(
                vobuf.at[0], outv_hbm.at[0, :, pl.ds(0, 128)], osem).wait()

        def do_pos(s, p):
            wait_idx(p)
            idxv = []
            for j in range(128 // L):
                v = ibuf[p, pl.ds(j * L, L)]
                idxv.append(jnp.minimum(jnp.maximum(v, 0), MAX_LEN))
            prefetch_idx(s + 2, p)

            @pl.when(s >= 2)
            def _drain():
                drain_out_pair()

            for d in range(D):
                base = jnp.int32(d * TW)
                for j in range(128 // L):
                    fidx = idxv[j] + base
                    kobuf[p, d, pl.ds(j * L, L)] = plsc.load_gather(
                        ktbl, [fidx])
                    vobuf[p, d, pl.ds(j * L, L)] = plsc.load_gather(
                        vtbl, [fidx])
            pltpu.async_copy(kobuf.at[p],
                             outk_hbm.at[s, :, pl.ds(b0, 128)], osem)
            pltpu.async_copy(vobuf.at[p],
                             outv_hbm.at[s, :, pl.ds(b0, 128)], osem)

        prefetch_idx(0, 0)
        prefetch_idx(1, 1)

        def body(i, carry):
            do_pos(2 * i, 0)
            do_pos(2 * i + 1, 1)
            return carry

        lax.fori_loop(0, S // 2, body, 0)
        wait_idx(0)
        wait_idx(1)
        drain_out_pair()
        drain_out_pair()

    return lookup


def _flat_table(pe):
    return jnp.pad(pe.T, ((0, 0), (0, TW - (MAX_LEN + 1)))).reshape(-1)


def kernel(position_mask, pe_k, pe_v):
    idx_t = position_mask.astype(jnp.int32).T
    out_k, out_v = _build_lookup()(
        idx_t, _flat_table(pe_k), _flat_table(pe_v))
    return (jnp.transpose(out_k, (2, 0, 1)), jnp.transpose(out_v, (2, 0, 1)))


# speedup vs baseline: 43.9375x; 4.5762x over previous
import functools

import jax
import jax.numpy as jnp
from jax import lax
from jax.experimental import pallas as pl
from jax.experimental.pallas import tpu as pltpu
from jax.experimental.pallas import tpu_sc as plsc

MAX_LEN = 200
D = 32
B = 4096
S = 200
L = 16
TW = 256


@functools.lru_cache(maxsize=None)
def _build_lookup():
    info = plsc.get_sparse_core_info()
    nc, ns = info.num_cores, info.num_subcores
    nw = nc * ns
    lanes_per_w = B // nw
    mesh = plsc.VectorSubcoreMesh(core_axis_name="c", subcore_axis_name="s")

    @functools.partial(
        pl.kernel,
        mesh=mesh,
        compiler_params=pltpu.CompilerParams(needs_layout_passes=False),
        out_type=[
            jax.ShapeDtypeStruct((S, D, B), jnp.float32),
            jax.ShapeDtypeStruct((S, D, B), jnp.float32),
        ],
        scratch_types=[
            pltpu.VMEM((D * TW,), jnp.float32),
            pltpu.VMEM((D * TW,), jnp.float32),
            pltpu.VMEM((2, 128), jnp.int32),
            pltpu.VMEM((2, D, 128), jnp.float32),
            pltpu.VMEM((2, D, 128), jnp.float32),
            pltpu.SemaphoreType.DMA,
            pltpu.SemaphoreType.DMA,
            pltpu.SemaphoreType.DMA,
        ],
    )
    def lookup(idx_hbm, tblk_hbm, tblv_hbm, outk_hbm, outv_hbm,
               ktbl, vtbl, ibuf, kobuf, vobuf, tsem, isem, osem):
        wid = lax.axis_index("s") * nc + lax.axis_index("c")
        b0 = wid * lanes_per_w

        ct = pltpu.async_copy(tblk_hbm, ktbl, tsem)
        cv = pltpu.async_copy(tblv_hbm, vtbl, tsem)
        ct.wait()
        cv.wait()

        def prefetch_idx(s, p):
            row = jnp.minimum(s, S - 1)
            pltpu.async_copy(idx_hbm.at[row, pl.ds(b0, 128)], ibuf.at[p],
                             isem)

        def wait_idx(p):
            pltpu.make_async_copy(
                idx_hbm.at[0, pl.ds(0, 128)], ibuf.at[p], isem).wait()

        def drain_out_pair():
            pltpu.make_async_copy(
                kobuf.at[0], outk_hbm.at[0, :, pl.ds(0, 128)], osem).wait()
            pltpu.make_async_copy(
                vobuf.at[0], outv_hbm.at[0, :, pl.ds(0, 128)], osem).wait()

        def do_pos(s, p):
            wait_idx(p)
            idxv = []
            for j in range(128 // L):
                v = ibuf[p, pl.ds(j * L, L)]
                idxv.append(jnp.minimum(jnp.maximum(v, 0), MAX_LEN))
            prefetch_idx(s + 2, p)

            @pl.when(s >= 2)
            def _drain():
                drain_out_pair()

            nj = 128 // L
            kv = [plsc.load_gather(ktbl, [idxv[j]]) for j in range(nj)]
            vv = [plsc.load_gather(vtbl, [idxv[j]]) for j in range(nj)]
            for d in range(1, D):
                base = jnp.int32(d * TW)
                nk, nv = [], []
                for j in range(nj):
                    f = idxv[j] + base
                    nk.append(plsc.load_gather(ktbl, [f]))
                    kobuf[p, d - 1, pl.ds(j * L, L)] = kv[j]
                    nv.append(plsc.load_gather(vtbl, [f]))
                    vobuf[p, d - 1, pl.ds(j * L, L)] = vv[j]
                kv, vv = nk, nv
            for j in range(nj):
                kobuf[p, D - 1, pl.ds(j * L, L)] = kv[j]
                vobuf[p, D - 1, pl.ds(j * L, L)] = vv[j]
            pltpu.async_copy(kobuf.at[p],
                             outk_hbm.at[s, :, pl.ds(b0, 128)], osem)
            pltpu.async_copy(vobuf.at[p],
                             outv_hbm.at[s, :, pl.ds(b0, 128)], osem)

        prefetch_idx(0, 0)
        prefetch_idx(1, 1)

        def body(i, carry):
            do_pos(2 * i, 0)
            do_pos(2 * i + 1, 1)
            return carry

        lax.fori_loop(0, S // 2, body, 0)
        wait_idx(0)
        wait_idx(1)
        drain_out_pair()
        drain_out_pair()

    return lookup


def _flat_table(pe):
    return jnp.pad(pe.T, ((0, 0), (0, TW - (MAX_LEN + 1)))).reshape(-1)


def kernel(position_mask, pe_k, pe_v):
    idx_t = position_mask.astype(jnp.int32).T
    out_k, out_v = _build_lookup()(
        idx_t, _flat_table(pe_k), _flat_table(pe_v))
    return (jnp.transpose(out_k, (2, 0, 1)), jnp.transpose(out_v, (2, 0, 1)))
